# Initial kernel scaffold; baseline (speedup 1.0000x reference)
import jax
import jax.numpy as jnp
from jax.experimental import pallas as pl


def kernel(inputs, embed_weight, W1, b1, W2, b2):
    raise NotImplementedError("write your pallas kernel here")


# speedup vs baseline: 1.1782x; 1.1782x over previous
import jax
import jax.numpy as jnp
from jax import lax
from jax.experimental import pallas as pl
from jax.experimental.pallas import tpu as pltpu
from jax.experimental.pallas import tpu_sc as plsc

VOCAB = 400000
D = 100
DP = 16
HID = 10
NCLS = 3
B = 4096
L = 200

NC = 2
NS = 16
NW = NC * NS
BPW = B // NW
CH = 20
NCHUNK = L // CH

PROJ_BLK = 8000


def _proj_body(emb_ref, w_ref, out_ref):
    out_ref[...] = jnp.dot(emb_ref[...], w_ref[...],
                           preferred_element_type=jnp.float32)


def _project(embed_weight, w1p):
    return pl.pallas_call(
        _proj_body,
        grid=(VOCAB // PROJ_BLK,),
        in_specs=[
            pl.BlockSpec((PROJ_BLK, D), lambda i: (i, 0)),
            pl.BlockSpec((D, DP), lambda i: (0, 0)),
        ],
        out_specs=pl.BlockSpec((PROJ_BLK, DP), lambda i: (i, 0)),
        out_shape=jax.ShapeDtypeStruct((VOCAB, DP), jnp.float32),
    )(embed_weight, w1p)


def _sc_body(p_hbm, idx_hbm, out_hbm, idx_v, rows_v, acc_v, sem):
    wid = lax.axis_index("c") * NS + lax.axis_index("s")

    pltpu.sync_copy(idx_hbm.at[wid], idx_v)

    zero = jnp.zeros((DP,), jnp.float32)

    def z_body(i, c):
        acc_v[i] = zero
        return c
    lax.fori_loop(0, BPW, z_body, 0)

    def chunk_body(g, c):
        base = g * CH
        copies = [
            pltpu.async_copy(p_hbm.at[idx_v.at[base + j]], rows_v.at[j], sem)
            for j in range(CH)
        ]
        for cp in copies:
            cp.wait()

        def item_body(i, cc):
            v = rows_v[0, i]
            for j in range(1, CH):
                v = v + rows_v[j, i]
            acc_v[i] = acc_v[i] + v
            return cc
        lax.fori_loop(0, BPW, item_body, 0)
        return c
    lax.fori_loop(0, NCHUNK, chunk_body, 0)

    pltpu.sync_copy(acc_v, out_hbm.at[wid])


_sc_call = pl.kernel(
    _sc_body,
    out_type=jax.ShapeDtypeStruct((NW, BPW, DP), jnp.float32),
    mesh=plsc.VectorSubcoreMesh(core_axis_name="c", subcore_axis_name="s",
                                num_cores=NC, num_subcores=NS),
    scratch_types=[
        pltpu.VMEM((L, BPW), jnp.int32),
        pltpu.VMEM((CH, BPW, DP), jnp.float32),
        pltpu.VMEM((BPW, DP), jnp.float32),
        pltpu.SemaphoreType.DMA,
    ],
    compiler_params=pltpu.CompilerParams(use_tc_tiling_on_sc=False),
)


def _mlp_body(s_ref, b1_ref, w2_ref, b2_ref, out_ref):
    h = jnp.maximum(s_ref[...] * jnp.float32(1.0 / L) + b1_ref[...], 0.0)
    out_ref[...] = jnp.dot(h, w2_ref[...],
                           preferred_element_type=jnp.float32) + b2_ref[...]


def _mlp(sums, b1p, w2p, b2p):
    return pl.pallas_call(
        _mlp_body,
        out_shape=jax.ShapeDtypeStruct((B, DP), jnp.float32),
    )(sums, b1p, w2p, b2p)


@jax.jit
def kernel(inputs, embed_weight, W1, b1, W2, b2):
    idx = inputs.astype(jnp.int32)
    w1p = jnp.zeros((D, DP), jnp.float32).at[:, :HID].set(W1.T)
    proj = _project(embed_weight, w1p)

    idx3 = idx.reshape(NW, BPW, L).transpose(0, 2, 1)

    sums = _sc_call(proj, idx3)

    b1p = jnp.zeros((1, DP), jnp.float32).at[0, :HID].set(b1)
    w2p = jnp.zeros((DP, DP), jnp.float32).at[:HID, :NCLS].set(W2.T)
    b2p = jnp.zeros((1, DP), jnp.float32).at[0, :NCLS].set(b2)

    out = _mlp(sums.reshape(B, DP), b1p, w2p, b2p)
    return out[:, :NCLS]


# speedup vs baseline: 1.1804x; 1.0019x over previous
import jax
import jax.numpy as jnp
from jax import lax
from jax.experimental import pallas as pl
from jax.experimental.pallas import tpu as pltpu
from jax.experimental.pallas import tpu_sc as plsc

VOCAB = 400000
D = 100
DP = 16
HID = 10
NCLS = 3
B = 4096
L = 200

NC = 2
NS = 16
NW = NC * NS
BPW = B // NW
CH = 20
NCHUNK = L // CH

PROJ_BLK = 20000


def _proj_body(emb_ref, w_ref, out_ref):
    out_ref[...] = jnp.dot(emb_ref[...], w_ref[...],
                           preferred_element_type=jnp.float32)


def _project(embed_weight, w1p):
    return pl.pallas_call(
        _proj_body,
        grid=(VOCAB // PROJ_BLK,),
        in_specs=[
            pl.BlockSpec((PROJ_BLK, D), lambda i: (i, 0)),
            pl.BlockSpec((D, DP), lambda i: (0, 0)),
        ],
        out_specs=pl.BlockSpec((PROJ_BLK, DP), lambda i: (i, 0)),
        out_shape=jax.ShapeDtypeStruct((VOCAB, DP), jnp.float32),
    )(embed_weight, w1p)


def _sc_body(p_hbm, idx_hbm, out_hbm, idx_v, rows_v, acc_v, sem):
    wid = lax.axis_index("c") * NS + lax.axis_index("s")

    pltpu.sync_copy(idx_hbm.at[wid], idx_v)

    zero = jnp.zeros((DP,), jnp.float32)

    def z_body(i, c):
        acc_v[i] = zero
        return c
    lax.fori_loop(0, BPW, z_body, 0)

    def chunk_body(g, c):
        base = g * CH
        copies = [
            pltpu.async_copy(p_hbm.at[idx_v.at[base + j]], rows_v.at[j], sem)
            for j in range(CH)
        ]
        for cp in copies:
            cp.wait()

        def item_body(i, cc):
            v = rows_v[0, i]
            for j in range(1, CH):
                v = v + rows_v[j, i]
            acc_v[i] = acc_v[i] + v
            return cc
        lax.fori_loop(0, BPW, item_body, 0)
        return c
    lax.fori_loop(0, NCHUNK, chunk_body, 0)

    pltpu.sync_copy(acc_v, out_hbm.at[wid])


_sc_call = pl.kernel(
    _sc_body,
    out_type=jax.ShapeDtypeStruct((NW, BPW, DP), jnp.float32),
    mesh=plsc.VectorSubcoreMesh(core_axis_name="c", subcore_axis_name="s",
                                num_cores=NC, num_subcores=NS),
    scratch_types=[
        pltpu.VMEM((L, BPW), jnp.int32),
        pltpu.VMEM((CH, BPW, DP), jnp.float32),
        pltpu.VMEM((BPW, DP), jnp.float32),
        pltpu.SemaphoreType.DMA,
    ],
    compiler_params=pltpu.CompilerParams(use_tc_tiling_on_sc=False),
)


def _mlp_body(s_ref, b1_ref, w2_ref, b2_ref, out_ref):
    h = jnp.maximum(s_ref[...] * jnp.float32(1.0 / L) + b1_ref[...], 0.0)
    out_ref[...] = jnp.dot(h, w2_ref[...],
                           preferred_element_type=jnp.float32) + b2_ref[...]


def _mlp(sums, b1p, w2p, b2p):
    return pl.pallas_call(
        _mlp_body,
        out_shape=jax.ShapeDtypeStruct((B, DP), jnp.float32),
    )(sums, b1p, w2p, b2p)


@jax.jit
def kernel(inputs, embed_weight, W1, b1, W2, b2):
    idx = inputs.astype(jnp.int32)
    w1p = jnp.zeros((D, DP), jnp.float32).at[:, :HID].set(W1.T)
    proj = _project(embed_weight, w1p)

    idx3 = idx.reshape(NW, BPW, L).transpose(0, 2, 1)

    sums = _sc_call(proj, idx3)

    b1p = jnp.zeros((1, DP), jnp.float32).at[0, :HID].set(b1)
    w2p = jnp.zeros((DP, DP), jnp.float32).at[:HID, :NCLS].set(W2.T)
    b2p = jnp.zeros((1, DP), jnp.float32).at[0, :NCLS].set(b2)

    out = _mlp(sums.reshape(B, DP), b1p, w2p, b2p)
    return out[:, :NCLS]
